# baseline (speedup 1.0000x reference)
import functools

import jax
import jax.numpy as jnp
from jax import lax
from jax.experimental import pallas as pl
from jax.experimental.pallas import tpu as pltpu
from jax.experimental.pallas import tpu_sc as plsc

FEAT = 128
NUM_K = 1024
POOL_PER = 256
TOTAL = NUM_K * POOL_PER
M = 16384

NUM_TILES = 32
CLUSTERS_PER_TILE = NUM_K // NUM_TILES
CAP = 120
IDXBUF = 128
STARTS_LEN = 48
STARTS_PAD = 1088
UPD_PAD = M + IDXBUF


def _sc_body(act_hbm, rows_hbm, scol_hbm, starts_hbm, pool_hbm, out_hbm,
             starts_v, rowid_v, scol_v, rows_v, block_v, sem):
    wid = lax.axis_index("c") * 16 + lax.axis_index("s")
    c0 = wid * CLUSTERS_PER_TILE

    pltpu.sync_copy(starts_hbm.at[pl.ds(c0, STARTS_LEN)], starts_v)

    def per_cluster(j, _):
        c = c0 + j
        col0 = c * POOL_PER
        pltpu.sync_copy(pool_hbm.at[:, pl.ds(col0, POOL_PER)], block_v)

        sv = starts_v[pl.ds(j, 16)]
        s = sv[0]
        e = sv[1]
        nchunks = (e - s + (CAP - 1)) // CAP

        def per_chunk(k2, _):
            base = s + k2 * CAP
            hi = jnp.minimum(base + CAP, e)
            a = (base // 8) * 8
            pltpu.sync_copy(rows_hbm.at[pl.ds(a, IDXBUF)], rowid_v)
            pltpu.sync_copy(scol_hbm.at[pl.ds(a, IDXBUF)],
                            scol_v.at[pl.ds(0, IDXBUF)])
            pltpu.async_copy(act_hbm.at[rowid_v], rows_v, sem).wait()

            def apply(p, _):
                q = p - a
                o = scol_v[pl.ds(q, 16)][0] - col0
                cidx = jnp.full((16,), o, dtype=jnp.int32)
                for fv in range(FEAT // 16):
                    vals = rows_v[q, pl.ds(fv * 16, 16)]
                    ridx = lax.iota(jnp.int32, 16) + fv * 16
                    plsc.store_scatter(block_v, [ridx, cidx], vals)
                return 0

            lax.fori_loop(base, hi, apply, 0)
            return 0

        lax.fori_loop(0, nchunks, per_chunk, 0)

        pltpu.sync_copy(block_v, out_hbm.at[:, pl.ds(col0, POOL_PER)])
        return 0

    lax.fori_loop(0, CLUSTERS_PER_TILE, per_cluster, 0)


def kernel(activation, cluster_num, rand_offsets, concept_pool):
    idx = (cluster_num.astype(jnp.int32) * POOL_PER
           + rand_offsets.astype(jnp.int32))
    order = jnp.argsort(idx, stable=True).astype(jnp.int32)
    scol = idx[order]
    bounds = jnp.arange(NUM_K + 1, dtype=jnp.int32) * POOL_PER
    starts = jnp.searchsorted(scol, bounds, side="left").astype(jnp.int32)
    starts_p = jnp.pad(starts, (0, STARTS_PAD - (NUM_K + 1)),
                       constant_values=M)
    rows_p = jnp.pad(order, (0, UPD_PAD - M))
    scol_p = jnp.pad(scol, (0, UPD_PAD - M))

    mesh = plsc.VectorSubcoreMesh(core_axis_name="c", subcore_axis_name="s",
                                  num_cores=2, num_subcores=16)
    run = pl.kernel(
        _sc_body,
        out_type=jax.ShapeDtypeStruct((FEAT, TOTAL), jnp.float32),
        mesh=mesh,
        scratch_types=[
            pltpu.VMEM((STARTS_LEN,), jnp.int32),
            pltpu.VMEM((IDXBUF,), jnp.int32),
            pltpu.VMEM((IDXBUF + 16,), jnp.int32),
            pltpu.VMEM((IDXBUF, FEAT), jnp.float32),
            pltpu.VMEM((FEAT, POOL_PER), jnp.float32),
            pltpu.SemaphoreType.DMA,
        ],
        compiler_params=pltpu.CompilerParams(use_tc_tiling_on_sc=True,
                                             needs_layout_passes=False),
    )
    return run(activation, rows_p, scol_p, starts_p, concept_pool)


# speedup vs baseline: 1.6197x; 1.6197x over previous
import jax
import jax.numpy as jnp
from jax import lax
from jax.experimental import pallas as pl
from jax.experimental.pallas import tpu as pltpu
from jax.experimental.pallas import tpu_sc as plsc

FEAT = 128
NUM_K = 1024
POOL_PER = 256
TOTAL = NUM_K * POOL_PER
M = 16384

NUM_TILES = 32
CPT = NUM_K // NUM_TILES
CAP = 40
IDXBUF = 48
SCOL_LEN = 64
STARTS_LEN = 48
STARTS_PAD = 1088
UPD_PAD = M + IDXBUF


def _sc_body(act_hbm, rows_hbm, scol_hbm, starts_hbm, pool_hbm, out_hbm,
             starts_v, rowid_v, scol_v, rows_v, blocks_v,
             lsem, ssem, isem, gsem):
    wid = lax.axis_index("c") * 16 + lax.axis_index("s")
    c0 = wid * CPT

    pltpu.sync_copy(starts_hbm.at[pl.ds(c0, STARTS_LEN)], starts_v)

    def cluster_start(j):
        return starts_v[pl.ds(j, 16)][0]

    def col_window(j):
        return pl.ds((c0 + j) * POOL_PER, POOL_PER)

    def start_idx(j, b):
        a = (cluster_start(j) // 8) * 8
        pltpu.async_copy(rows_hbm.at[pl.ds(a, IDXBUF)], rowid_v.at[b],
                         isem.at[b])
        pltpu.async_copy(scol_hbm.at[pl.ds(a, IDXBUF)],
                         scol_v.at[b].at[pl.ds(0, IDXBUF)], isem.at[b])

    def wait_idx(b):
        pltpu.make_async_copy(rows_hbm.at[pl.ds(0, IDXBUF)], rowid_v.at[b],
                              isem.at[b]).wait()
        pltpu.make_async_copy(scol_hbm.at[pl.ds(0, IDXBUF)],
                              scol_v.at[b].at[pl.ds(0, IDXBUF)],
                              isem.at[b]).wait()

    def issue_gather(b):
        pltpu.async_copy(act_hbm.at[rowid_v.at[b]], rows_v.at[b], gsem.at[b])

    def wait_gather(b):
        pltpu.make_async_copy(act_hbm.at[rowid_v.at[b]], rows_v.at[b],
                              gsem.at[b]).wait()

    def start_load(j, b):
        pltpu.async_copy(pool_hbm.at[:, col_window(j)], blocks_v.at[b],
                         lsem.at[b])

    def wait_load(j, b):
        pltpu.make_async_copy(pool_hbm.at[:, col_window(j)], blocks_v.at[b],
                              lsem.at[b]).wait()

    def start_store(j, b):
        pltpu.async_copy(blocks_v.at[b], out_hbm.at[:, col_window(j)],
                         ssem.at[b])

    def wait_store(b):
        pltpu.make_async_copy(blocks_v.at[b], out_hbm.at[:, col_window(0)],
                              ssem.at[b]).wait()

    def apply_range(lo, hi, a, b, col0):
        def apply(p, _):
            q = p - a
            o = scol_v[b, pl.ds(q, 16)][0] - col0
            cidx = jnp.full((16,), o, dtype=jnp.int32)
            for fv in range(FEAT // 16):
                vals = rows_v[b, q, pl.ds(fv * 16, 16)]
                ridx = lax.iota(jnp.int32, 16) + fv * 16
                plsc.store_scatter(blocks_v.at[b], [ridx, cidx], vals)
            return 0

        lax.fori_loop(lo, hi, apply, 0)

    def apply_cluster(j, b):
        col0 = (c0 + j) * POOL_PER
        sv = starts_v[pl.ds(j, 16)]
        s = sv[0]
        e = sv[1]
        apply_range(s, jnp.minimum(s + CAP, e), (s // 8) * 8, b, col0)
        nch = (e - s + (CAP - 1)) // CAP

        def rare(k, _):
            base = s + k * CAP
            a = (base // 8) * 8
            pltpu.sync_copy(rows_hbm.at[pl.ds(a, IDXBUF)], rowid_v.at[b])
            pltpu.sync_copy(scol_hbm.at[pl.ds(a, IDXBUF)],
                            scol_v.at[b].at[pl.ds(0, IDXBUF)])
            pltpu.async_copy(act_hbm.at[rowid_v.at[b]], rows_v.at[b],
                             gsem.at[b]).wait()
            apply_range(base, jnp.minimum(base + CAP, e), a, b, col0)
            return 0

        lax.fori_loop(1, nch, rare, 0)

    def step(j, r):
        nb = (r + 1) % 3
        pb = (r + 2) % 3

        @pl.when(j >= 2)
        def _():
            wait_store(nb)

        start_load(j + 1, nb)
        start_idx(j + 2, pb)
        wait_idx(nb)
        issue_gather(nb)
        wait_load(j, r)
        wait_gather(r)
        apply_cluster(j, r)
        start_store(j, r)

    start_idx(0, 0)
    wait_idx(0)
    issue_gather(0)
    start_idx(1, 1)
    start_load(0, 0)

    def loop(i, _):
        for r in range(3):
            step(3 * i + r, r)
        return 0

    lax.fori_loop(0, 10, loop, 0)

    wait_store(1)
    start_load(31, 1)
    wait_idx(1)
    issue_gather(1)
    wait_load(30, 0)
    wait_gather(0)
    apply_cluster(30, 0)
    start_store(30, 0)

    wait_load(31, 1)
    wait_gather(1)
    apply_cluster(31, 1)
    start_store(31, 1)

    wait_store(0)
    wait_store(1)
    wait_store(2)


def kernel(activation, cluster_num, rand_offsets, concept_pool):
    idx = (cluster_num.astype(jnp.int32) * POOL_PER
           + rand_offsets.astype(jnp.int32))
    order = jnp.argsort(idx, stable=True).astype(jnp.int32)
    scol = idx[order]
    bounds = jnp.arange(NUM_K + 1, dtype=jnp.int32) * POOL_PER
    starts = jnp.searchsorted(scol, bounds, side="left").astype(jnp.int32)
    starts_p = jnp.pad(starts, (0, STARTS_PAD - (NUM_K + 1)),
                       constant_values=M)
    rows_p = jnp.pad(order, (0, UPD_PAD - M))
    scol_p = jnp.pad(scol, (0, UPD_PAD - M))

    mesh = plsc.VectorSubcoreMesh(core_axis_name="c", subcore_axis_name="s",
                                  num_cores=2, num_subcores=16)
    run = pl.kernel(
        _sc_body,
        out_type=jax.ShapeDtypeStruct((FEAT, TOTAL), jnp.float32),
        mesh=mesh,
        scratch_types=[
            pltpu.VMEM((STARTS_LEN,), jnp.int32),
            pltpu.VMEM((3, IDXBUF), jnp.int32),
            pltpu.VMEM((3, SCOL_LEN), jnp.int32),
            pltpu.VMEM((3, IDXBUF, FEAT), jnp.float32),
            pltpu.VMEM((3, FEAT, POOL_PER), jnp.float32),
            pltpu.SemaphoreType.DMA((3,)),
            pltpu.SemaphoreType.DMA((3,)),
            pltpu.SemaphoreType.DMA((3,)),
            pltpu.SemaphoreType.DMA((3,)),
        ],
        compiler_params=pltpu.CompilerParams(use_tc_tiling_on_sc=True,
                                             needs_layout_passes=False),
    )
    return run(activation, rows_p, scol_p, starts_p, concept_pool)
